# baseline (speedup 1.0000x reference)
import jax
import jax.numpy as jnp
from jax.experimental import pallas as pl
from jax.experimental.pallas import tpu as pltpu

T = 128
H = 1024
E = 64
I = 512
EPB = 1


def _moe_body(x_ref, wg_ref, bias_ref, wgu_a_ref, wgu_b_ref, wgu_c_ref,
              wgu_d_ref, wd_a_ref, wd_b_ref, wsgu_ref, wsd_ref, o_ref,
              combine_ref):
    e = pl.program_id(0)
    x = x_ref[...]

    @pl.when(e == 0)
    def _router_and_shared():
        logits = jnp.dot(x, wg_ref[...], preferred_element_type=jnp.float32)
        s = jax.nn.sigmoid(logits)
        choice = s + bias_ref[...]
        cols = jax.lax.broadcasted_iota(jnp.int32, (T, E), 1)
        idx1 = jnp.argmax(choice, axis=1)
        m1 = cols == idx1[:, None]
        choice2 = jnp.where(m1, -jnp.inf, choice)
        idx2 = jnp.argmax(choice2, axis=1)
        m2 = cols == idx2[:, None]
        w1 = jnp.sum(jnp.where(m1, s, 0.0), axis=1)
        w2 = jnp.sum(jnp.where(m2, s, 0.0), axis=1)
        inv = 1.0 / (w1 + w2)
        combine_ref[...] = (jnp.where(m1, (w1 * inv)[:, None], 0.0) +
                            jnp.where(m2, (w2 * inv)[:, None], 0.0))
        gu = jnp.dot(x, wsgu_ref[...], preferred_element_type=jnp.float32)
        act = jax.nn.silu(gu[:, :I]) * gu[:, I:]
        o_ref[...] = jnp.dot(act, wsd_ref[...], preferred_element_type=jnp.float32)

    xb = x.astype(jnp.bfloat16)
    cols = jax.lax.broadcasted_iota(jnp.int32, (T, E), 1)
    acc = jnp.zeros((T, H), jnp.float32)
    for j in range(EPB):
        gate_a = jnp.dot(xb, wgu_a_ref[j].astype(jnp.bfloat16),
                         preferred_element_type=jnp.float32)
        gate_b = jnp.dot(xb, wgu_b_ref[j].astype(jnp.bfloat16),
                         preferred_element_type=jnp.float32)
        up_a = jnp.dot(xb, wgu_c_ref[j].astype(jnp.bfloat16),
                       preferred_element_type=jnp.float32)
        up_b = jnp.dot(xb, wgu_d_ref[j].astype(jnp.bfloat16),
                       preferred_element_type=jnp.float32)
        act_a = (jax.nn.silu(gate_a) * up_a).astype(jnp.bfloat16)
        act_b = (jax.nn.silu(gate_b) * up_b).astype(jnp.bfloat16)
        oe = (jnp.dot(act_a, wd_a_ref[j].astype(jnp.bfloat16),
                      preferred_element_type=jnp.float32) +
              jnp.dot(act_b, wd_b_ref[j].astype(jnp.bfloat16),
                      preferred_element_type=jnp.float32))
        w_e = jnp.sum(jnp.where(cols == e * EPB + j, combine_ref[...], 0.0),
                      axis=1, keepdims=True)
        acc += w_e * oe
    o_ref[...] += acc


def kernel(hidden_states, Wg, Wgu, Wd, Ws_gu, Ws_d, expert_bias):
    bias2d = expert_bias.reshape(1, E)
    return pl.pallas_call(
        _moe_body,
        grid=(E // EPB,),
        in_specs=[
            pl.BlockSpec((T, H), lambda e: (0, 0)),
            pl.BlockSpec((H, E), lambda e: (0, 0)),
            pl.BlockSpec((1, E), lambda e: (0, 0)),
            pl.BlockSpec((EPB, H, I // 2), lambda e: (e, 0, 0)),
            pl.BlockSpec((EPB, H, I // 2), lambda e: (e, 0, 1)),
            pl.BlockSpec((EPB, H, I // 2), lambda e: (e, 0, 2)),
            pl.BlockSpec((EPB, H, I // 2), lambda e: (e, 0, 3)),
            pl.BlockSpec((EPB, I // 2, H), lambda e: (e, 0, 0)),
            pl.BlockSpec((EPB, I // 2, H), lambda e: (e, 1, 0)),
            pl.BlockSpec((H, 2 * I), lambda e: (0, 0)),
            pl.BlockSpec((I, H), lambda e: (0, 0)),
        ],
        out_specs=pl.BlockSpec((T, H), lambda e: (0, 0)),
        out_shape=jax.ShapeDtypeStruct((T, H), jnp.float32),
        scratch_shapes=[pltpu.VMEM((T, E), jnp.float32)],
    )(hidden_states, Wg, bias2d, Wgu, Wgu, Wgu, Wgu, Wd, Wd, Ws_gu, Ws_d)


# speedup vs baseline: 1.0473x; 1.0473x over previous
import jax
import jax.numpy as jnp
from jax.experimental import pallas as pl
from jax.experimental.pallas import tpu as pltpu

T = 128
H = 1024
E = 64
I = 512
EPB = 2


def _moe_body(x_ref, wg_ref, bias_ref, wgu_a_ref, wgu_b_ref, wgu_c_ref,
              wgu_d_ref, wd_a_ref, wd_b_ref, wsgu_ref, wsd_ref, o_ref,
              combine_ref):
    e = pl.program_id(0)
    x = x_ref[...]

    @pl.when(e == 0)
    def _router_and_shared():
        logits = jnp.dot(x, wg_ref[...], preferred_element_type=jnp.float32)
        s = jax.nn.sigmoid(logits)
        choice = s + bias_ref[...]
        cols = jax.lax.broadcasted_iota(jnp.int32, (T, E), 1)
        idx1 = jnp.argmax(choice, axis=1)
        m1 = cols == idx1[:, None]
        choice2 = jnp.where(m1, -jnp.inf, choice)
        idx2 = jnp.argmax(choice2, axis=1)
        m2 = cols == idx2[:, None]
        w1 = jnp.sum(jnp.where(m1, s, 0.0), axis=1)
        w2 = jnp.sum(jnp.where(m2, s, 0.0), axis=1)
        inv = 1.0 / (w1 + w2)
        combine_ref[...] = (jnp.where(m1, (w1 * inv)[:, None], 0.0) +
                            jnp.where(m2, (w2 * inv)[:, None], 0.0))
        gu = jnp.dot(x, wsgu_ref[...], preferred_element_type=jnp.float32)
        act = jax.nn.silu(gu[:, :I]) * gu[:, I:]
        o_ref[...] = jnp.dot(act, wsd_ref[...], preferred_element_type=jnp.float32)

    xb = x.astype(jnp.bfloat16)
    cols = jax.lax.broadcasted_iota(jnp.int32, (T, E), 1)
    acc = jnp.zeros((T, H), jnp.float32)
    for j in range(EPB):
        gate_a = jnp.dot(xb, wgu_a_ref[j].astype(jnp.bfloat16),
                         preferred_element_type=jnp.float32)
        gate_b = jnp.dot(xb, wgu_b_ref[j].astype(jnp.bfloat16),
                         preferred_element_type=jnp.float32)
        up_a = jnp.dot(xb, wgu_c_ref[j].astype(jnp.bfloat16),
                       preferred_element_type=jnp.float32)
        up_b = jnp.dot(xb, wgu_d_ref[j].astype(jnp.bfloat16),
                       preferred_element_type=jnp.float32)
        act_a = (jax.nn.silu(gate_a) * up_a).astype(jnp.bfloat16)
        act_b = (jax.nn.silu(gate_b) * up_b).astype(jnp.bfloat16)
        oe = (jnp.dot(act_a, wd_a_ref[j].astype(jnp.bfloat16),
                      preferred_element_type=jnp.float32) +
              jnp.dot(act_b, wd_b_ref[j].astype(jnp.bfloat16),
                      preferred_element_type=jnp.float32))
        w_e = jnp.sum(jnp.where(cols == e * EPB + j, combine_ref[...], 0.0),
                      axis=1, keepdims=True)
        acc += w_e * oe
    o_ref[...] += acc


def kernel(hidden_states, Wg, Wgu, Wd, Ws_gu, Ws_d, expert_bias):
    bias2d = expert_bias.reshape(1, E)
    return pl.pallas_call(
        _moe_body,
        grid=(E // EPB,),
        in_specs=[
            pl.BlockSpec((T, H), lambda e: (0, 0)),
            pl.BlockSpec((H, E), lambda e: (0, 0)),
            pl.BlockSpec((1, E), lambda e: (0, 0)),
            pl.BlockSpec((EPB, H, I // 2), lambda e: (e, 0, 0)),
            pl.BlockSpec((EPB, H, I // 2), lambda e: (e, 0, 1)),
            pl.BlockSpec((EPB, H, I // 2), lambda e: (e, 0, 2)),
            pl.BlockSpec((EPB, H, I // 2), lambda e: (e, 0, 3)),
            pl.BlockSpec((EPB, I // 2, H), lambda e: (e, 0, 0)),
            pl.BlockSpec((EPB, I // 2, H), lambda e: (e, 1, 0)),
            pl.BlockSpec((H, 2 * I), lambda e: (0, 0)),
            pl.BlockSpec((I, H), lambda e: (0, 0)),
        ],
        out_specs=pl.BlockSpec((T, H), lambda e: (0, 0)),
        out_shape=jax.ShapeDtypeStruct((T, H), jnp.float32),
        scratch_shapes=[pltpu.VMEM((T, E), jnp.float32)],
    )(hidden_states, Wg, bias2d, Wgu, Wgu, Wgu, Wgu, Wd, Wd, Ws_gu, Ws_d)
